# baseline (speedup 1.0000x reference)
import jax
import jax.numpy as jnp
from jax.experimental import pallas as pl
from jax.experimental.pallas import tpu as pltpu

_NUM_CLASSES = 81
_EPS = 1e-3
_NB = 400


def _bbox_head_kernel(
    a_lo_ref, a_hi_ref, w0_ref, w1_ref, w2q_ref, w3q_ref,
    wdense_ref, wct_ref, wdt_ref,
    b1_ref, g1_ref, be1_ref, b2_ref, g2_ref, be2_ref, bc_ref, bd_ref,
    logits_ref, probs_ref, deltas_ref,
    acc_ref,
):
    i = pl.program_id(0)
    n = pl.program_id(1)
    w_refs = (w0_ref, w1_ref, w2q_ref, w3q_ref)
    half = _NB // 2

    for h, a_ref in enumerate((a_lo_ref, a_hi_ref)):
        a = [a_ref[0, jj].astype(jnp.bfloat16) for jj in range(7)]
        row0 = pl.multiple_of(n * _NB, 8) + h * half
        rows = pl.ds(row0, half)
        for p in range(4):
            part = None
            for jj in range(7):
                w = w_refs[p][0, jj].astype(jnp.bfloat16)
                d = jnp.dot(a[jj], w, preferred_element_type=jnp.float32)
                part = d if part is None else part + d
            cols = pl.ds(256 * p, 256)

            @pl.when(i == 0)
            def _init(part=part, rows=rows, cols=cols):
                acc_ref[rows, cols] = part

            @pl.when(i > 0)
            def _accum(part=part, rows=rows, cols=cols):
                acc_ref[rows, cols] += part

    @pl.when((i == 6) & (n == pl.num_programs(1) - 1))
    def _epilogue():
        x1 = acc_ref[...] + b1_ref[...]
        mean1 = jnp.mean(x1, axis=0, keepdims=True)
        var1 = jnp.mean((x1 - mean1) ** 2, axis=0, keepdims=True)
        h1 = g1_ref[...] * (x1 - mean1) / jnp.sqrt(var1 + _EPS) + be1_ref[...]
        h1 = jnp.maximum(h1, 0.0).astype(jnp.bfloat16)

        w2 = wdense_ref[...].astype(jnp.bfloat16)
        x2 = jnp.dot(h1, w2, preferred_element_type=jnp.float32) + b2_ref[...]
        mean2 = jnp.mean(x2, axis=0, keepdims=True)
        var2 = jnp.mean((x2 - mean2) ** 2, axis=0, keepdims=True)
        h2 = g2_ref[...] * (x2 - mean2) / jnp.sqrt(var2 + _EPS) + be2_ref[...]
        h2 = jnp.maximum(h2, 0.0).astype(jnp.bfloat16)

        dims = (((1,), (1,)), ((), ()))
        wct = wct_ref[...].astype(jnp.bfloat16)
        logits = jax.lax.dot_general(
            h2, wct, dims, preferred_element_type=jnp.float32) + bc_ref[...]
        logits_ref[...] = logits
        m = jnp.max(logits, axis=1, keepdims=True)
        e = jnp.exp(logits - m)
        probs_ref[...] = e / jnp.sum(e, axis=1, keepdims=True)

        wdt = wdt_ref[...].astype(jnp.bfloat16)
        deltas_ref[...] = jax.lax.dot_general(
            h2, wdt, dims, preferred_element_type=jnp.float32) + bd_ref[...]


def kernel(pooled_rois, W1, b1, gamma1, beta1, W2, b2, gamma2, beta2, Wc, bc, Wd, bd):
    n = pooled_rois.shape[0]
    a_t = jnp.transpose(pooled_rois, (1, 2, 0, 3))
    wct = jnp.transpose(Wc)
    wdt = jnp.transpose(Wd)
    nc = Wc.shape[1]
    nd = Wd.shape[1]
    num_nb = n // _NB
    half = _NB // 2

    row = lambda v: v.reshape(1, -1)
    full = lambda arr: pl.BlockSpec(arr.shape, lambda i, j: (0,) * arr.ndim)

    logits, probs, deltas = pl.pallas_call(
        _bbox_head_kernel,
        grid=(7, num_nb),
        in_specs=[
            pl.BlockSpec((1, 7, half, 256), lambda i, j: (i, 0, 2 * j, 0)),
            pl.BlockSpec((1, 7, half, 256), lambda i, j: (i, 0, 2 * j + 1, 0)),
            pl.BlockSpec((1, 7, 256, 256), lambda i, j: (i, 0, 0, 0)),
            pl.BlockSpec((1, 7, 256, 256), lambda i, j: (i, 0, 0, 1)),
            pl.BlockSpec((1, 7, 256, 256), lambda i, j: (i, 0, 0, 2)),
            pl.BlockSpec((1, 7, 256, 256), lambda i, j: (i, 0, 0, 3)),
            full(W2), full(wct), full(wdt),
            full(row(b1)), full(row(gamma1)), full(row(beta1)),
            full(row(b2)), full(row(gamma2)), full(row(beta2)),
            full(row(bc)), full(row(bd)),
        ],
        out_specs=[
            pl.BlockSpec((n, nc), lambda i, j: (0, 0)),
            pl.BlockSpec((n, nc), lambda i, j: (0, 0)),
            pl.BlockSpec((n, nd), lambda i, j: (0, 0)),
        ],
        out_shape=[
            jax.ShapeDtypeStruct((n, nc), jnp.float32),
            jax.ShapeDtypeStruct((n, nc), jnp.float32),
            jax.ShapeDtypeStruct((n, nd), jnp.float32),
        ],
        scratch_shapes=[pltpu.VMEM((n, 1024), jnp.float32)],
        compiler_params=pltpu.CompilerParams(
            dimension_semantics=("arbitrary", "arbitrary"),
        ),
    )(
        a_t, a_t, W1, W1, W1, W1, W2, wct, wdt,
        row(b1), row(gamma1), row(beta1),
        row(b2), row(gamma2), row(beta2),
        row(bc), row(bd),
    )
    return (logits, probs, deltas.reshape(n, _NUM_CLASSES, 4))


# speedup vs baseline: 1.7996x; 1.7996x over previous
import jax
import jax.numpy as jnp
from jax.experimental import pallas as pl
from jax.experimental.pallas import tpu as pltpu

_NUM_CLASSES = 81
_EPS = 1e-3
_NB = 400


def _bbox_head_kernel(
    a_ref, w1_ref, wdense_ref, wct_ref, wdt_ref,
    b1_ref, g1_ref, be1_ref, b2_ref, g2_ref, be2_ref, bc_ref, bd_ref,
    logits_ref, probs_ref, deltas_ref,
    acc_ref,
):
    i = pl.program_id(0)
    n = pl.program_id(1)
    part = None
    for jj in range(7):
        a = a_ref[0, jj].astype(jnp.bfloat16)
        w = w1_ref[0, jj].astype(jnp.bfloat16)
        d = jnp.dot(a, w, preferred_element_type=jnp.float32)
        part = d if part is None else part + d
    rows = pl.ds(pl.multiple_of(n * _NB, 8), _NB)

    @pl.when(i == 0)
    def _init():
        acc_ref[rows, :] = part

    @pl.when(i > 0)
    def _accum():
        acc_ref[rows, :] += part

    @pl.when((i == 6) & (n == pl.num_programs(1) - 1))
    def _epilogue():
        x1 = acc_ref[...] + b1_ref[...]
        mean1 = jnp.mean(x1, axis=0, keepdims=True)
        var1 = jnp.mean((x1 - mean1) ** 2, axis=0, keepdims=True)
        h1 = g1_ref[...] * (x1 - mean1) / jnp.sqrt(var1 + _EPS) + be1_ref[...]
        h1 = jnp.maximum(h1, 0.0).astype(jnp.bfloat16)

        w2 = wdense_ref[...].astype(jnp.bfloat16)
        x2 = jnp.dot(h1, w2, preferred_element_type=jnp.float32) + b2_ref[...]
        mean2 = jnp.mean(x2, axis=0, keepdims=True)
        var2 = jnp.mean((x2 - mean2) ** 2, axis=0, keepdims=True)
        h2 = g2_ref[...] * (x2 - mean2) / jnp.sqrt(var2 + _EPS) + be2_ref[...]
        h2 = jnp.maximum(h2, 0.0).astype(jnp.bfloat16)

        dims = (((1,), (1,)), ((), ()))
        wct = wct_ref[...].astype(jnp.bfloat16)
        logits = jax.lax.dot_general(
            h2, wct, dims, preferred_element_type=jnp.float32) + bc_ref[...]
        logits_ref[...] = logits
        m = jnp.max(logits, axis=1, keepdims=True)
        e = jnp.exp(logits - m)
        probs_ref[...] = e / jnp.sum(e, axis=1, keepdims=True)

        wdt = wdt_ref[...].astype(jnp.bfloat16)
        deltas_ref[...] = jax.lax.dot_general(
            h2, wdt, dims, preferred_element_type=jnp.float32) + bd_ref[...]


def kernel(pooled_rois, W1, b1, gamma1, beta1, W2, b2, gamma2, beta2, Wc, bc, Wd, bd):
    n = pooled_rois.shape[0]
    a_t = jnp.transpose(pooled_rois, (1, 2, 0, 3))
    wct = jnp.transpose(Wc)
    wdt = jnp.transpose(Wd)
    nc = Wc.shape[1]
    nd = Wd.shape[1]
    num_nb = n // _NB

    row = lambda v: v.reshape(1, -1)
    full = lambda arr: pl.BlockSpec(arr.shape, lambda i, j: (0,) * arr.ndim)

    logits, probs, deltas = pl.pallas_call(
        _bbox_head_kernel,
        grid=(7, num_nb),
        in_specs=[
            pl.BlockSpec((1, 7, _NB, 256), lambda i, j: (i, 0, j, 0)),
            pl.BlockSpec((1, 7, 256, 1024), lambda i, j: (i, 0, 0, 0)),
            full(W2), full(wct), full(wdt),
            full(row(b1)), full(row(gamma1)), full(row(beta1)),
            full(row(b2)), full(row(gamma2)), full(row(beta2)),
            full(row(bc)), full(row(bd)),
        ],
        out_specs=[
            pl.BlockSpec((n, nc), lambda i, j: (0, 0)),
            pl.BlockSpec((n, nc), lambda i, j: (0, 0)),
            pl.BlockSpec((n, nd), lambda i, j: (0, 0)),
        ],
        out_shape=[
            jax.ShapeDtypeStruct((n, nc), jnp.float32),
            jax.ShapeDtypeStruct((n, nc), jnp.float32),
            jax.ShapeDtypeStruct((n, nd), jnp.float32),
        ],
        scratch_shapes=[pltpu.VMEM((n, 1024), jnp.float32)],
        compiler_params=pltpu.CompilerParams(
            dimension_semantics=("arbitrary", "arbitrary"),
        ),
    )(
        a_t, W1, W2, wct, wdt,
        row(b1), row(gamma1), row(beta1),
        row(b2), row(gamma2), row(beta2),
        row(bc), row(bd),
    )
    return (logits, probs, deltas.reshape(n, _NUM_CLASSES, 4))


# speedup vs baseline: 2.0850x; 1.1586x over previous
import jax
import jax.numpy as jnp
from jax.experimental import pallas as pl
from jax.experimental.pallas import tpu as pltpu

_NUM_CLASSES = 81
_EPS = 1e-3
_NB = 1000


def _bbox_head_kernel(
    a0_ref, a1_ref, a2_ref, a3_ref, a4_ref,
    w1_ref, wdense_ref, wct_ref, wdt_ref,
    g1_ref, be1_ref, g2_ref, be2_ref, bc_ref, bd_ref,
    logits_ref, probs_ref, deltas_ref,
    acc_ref, h_ref,
):
    i = pl.program_id(0)
    n = pl.program_id(1)
    a_refs = (a0_ref, a1_ref, a2_ref, a3_ref, a4_ref)
    part = None
    for jj in range(7):
        a = jnp.concatenate(
            [r[0, jj].astype(jnp.bfloat16) for r in a_refs], axis=0)
        w = w1_ref[0, jj].astype(jnp.bfloat16)
        d = jnp.dot(a, w, preferred_element_type=jnp.float32)
        part = d if part is None else part + d
    rows = pl.ds(pl.multiple_of(n * _NB, 8), _NB)

    @pl.when(i == 0)
    def _init():
        acc_ref[rows, :] = part

    @pl.when(i > 0)
    def _accum():
        acc_ref[rows, :] += part

    @pl.when((i == 6) & (n == pl.num_programs(1) - 1))
    def _epilogue():
        nrows = acc_ref.shape[0]
        nchunks = 5
        ch = nrows // nchunks
        inv_n = 1.0 / nrows

        def _bn_relu_to_h(scale, shift):
            for c in range(nchunks):
                r = pl.ds(c * ch, ch)
                h_ref[r, :] = jnp.maximum(
                    acc_ref[r, :] * scale + shift, 0.0
                ).astype(jnp.bfloat16)

        def _stats():
            s = None
            for c in range(nchunks):
                cs = jnp.sum(acc_ref[pl.ds(c * ch, ch), :], axis=0,
                             keepdims=True)
                s = cs if s is None else s + cs
            mean = s * inv_n
            v = None
            for c in range(nchunks):
                cv = jnp.sum(
                    (acc_ref[pl.ds(c * ch, ch), :] - mean) ** 2,
                    axis=0, keepdims=True)
                v = cv if v is None else v + cv
            return mean, v * inv_n

        mean1, var1 = _stats()
        scale1 = g1_ref[...] / jnp.sqrt(var1 + _EPS)
        _bn_relu_to_h(scale1, be1_ref[...] - mean1 * scale1)

        w2 = wdense_ref[...].astype(jnp.bfloat16)
        for c in range(2):
            r = pl.ds(c * (nrows // 2), nrows // 2)
            acc_ref[r, :] = jnp.dot(
                h_ref[r, :], w2, preferred_element_type=jnp.float32)

        mean2, var2 = _stats()
        scale2 = g2_ref[...] / jnp.sqrt(var2 + _EPS)
        _bn_relu_to_h(scale2, be2_ref[...] - mean2 * scale2)

        dims = (((1,), (1,)), ((), ()))
        wct = wct_ref[...].astype(jnp.bfloat16)
        wdt = wdt_ref[...].astype(jnp.bfloat16)
        for c in range(2):
            r = pl.ds(c * (nrows // 2), nrows // 2)
            h2 = h_ref[r, :]
            logits = jax.lax.dot_general(
                h2, wct, dims, preferred_element_type=jnp.float32
            ) + bc_ref[...]
            logits_ref[r, :] = logits
            m = jnp.max(logits, axis=1, keepdims=True)
            e = jnp.exp(logits - m)
            probs_ref[r, :] = e / jnp.sum(e, axis=1, keepdims=True)
            deltas_ref[r, :] = jax.lax.dot_general(
                h2, wdt, dims, preferred_element_type=jnp.float32
            ) + bd_ref[...]


def kernel(pooled_rois, W1, b1, gamma1, beta1, W2, b2, gamma2, beta2, Wc, bc, Wd, bd):
    n = pooled_rois.shape[0]
    a_t = jnp.transpose(pooled_rois, (1, 2, 0, 3))
    wct = jnp.transpose(Wc)
    wdt = jnp.transpose(Wd)
    nc = Wc.shape[1]
    nd = Wd.shape[1]
    num_nb = n // _NB

    row = lambda v: v.reshape(1, -1)
    full = lambda arr: pl.BlockSpec(arr.shape, lambda i, j: (0,) * arr.ndim)

    logits, probs, deltas = pl.pallas_call(
        _bbox_head_kernel,
        grid=(7, num_nb),
        in_specs=[
            pl.BlockSpec((1, 7, 200, 256), lambda i, j: (i, 0, 5 * j, 0)),
            pl.BlockSpec((1, 7, 200, 256), lambda i, j: (i, 0, 5 * j + 1, 0)),
            pl.BlockSpec((1, 7, 200, 256), lambda i, j: (i, 0, 5 * j + 2, 0)),
            pl.BlockSpec((1, 7, 200, 256), lambda i, j: (i, 0, 5 * j + 3, 0)),
            pl.BlockSpec((1, 7, 200, 256), lambda i, j: (i, 0, 5 * j + 4, 0)),
            pl.BlockSpec((1, 7, 256, 1024), lambda i, j: (i, 0, 0, 0)),
            full(W2), full(wct), full(wdt),
            full(row(gamma1)), full(row(beta1)),
            full(row(gamma2)), full(row(beta2)),
            full(row(bc)), full(row(bd)),
        ],
        out_specs=[
            pl.BlockSpec((n, nc), lambda i, j: (0, 0)),
            pl.BlockSpec((n, nc), lambda i, j: (0, 0)),
            pl.BlockSpec((n, nd), lambda i, j: (0, 0)),
        ],
        out_shape=[
            jax.ShapeDtypeStruct((n, nc), jnp.float32),
            jax.ShapeDtypeStruct((n, nc), jnp.float32),
            jax.ShapeDtypeStruct((n, nd), jnp.float32),
        ],
        scratch_shapes=[
            pltpu.VMEM((n, 1024), jnp.float32),
            pltpu.VMEM((n, 1024), jnp.bfloat16),
        ],
        compiler_params=pltpu.CompilerParams(
            dimension_semantics=("arbitrary", "arbitrary"),
        ),
    )(
        a_t, a_t, a_t, a_t, a_t, W1, W2, wct, wdt,
        row(gamma1), row(beta1), row(gamma2), row(beta2),
        row(bc), row(bd),
    )
    return (logits, probs, deltas.reshape(n, _NUM_CLASSES, 4))


# speedup vs baseline: 2.1964x; 1.0534x over previous
import jax
import jax.numpy as jnp
from jax.experimental import pallas as pl
from jax.experimental.pallas import tpu as pltpu

_NUM_CLASSES = 81
_EPS = 1e-3
_NB = 1000


def _bbox_head_kernel(
    a0_ref, a1_ref, a2_ref, a3_ref, a4_ref,
    w1_ref, wdense_ref, wct_ref, wdt_ref,
    g1_ref, be1_ref, g2_ref, be2_ref, bc_ref, bd_ref,
    logits_ref, probs_ref, deltas_ref,
    acc_ref, h_ref,
):
    i = pl.program_id(0)
    n = pl.program_id(1)
    a_refs = (a0_ref, a1_ref, a2_ref, a3_ref, a4_ref)
    part = None
    for jj in range(7):
        a = jnp.concatenate(
            [r[0, jj].astype(jnp.bfloat16) for r in a_refs], axis=0)
        w = w1_ref[0, jj].astype(jnp.bfloat16)
        d = jnp.dot(a, w, preferred_element_type=jnp.float32)
        part = d if part is None else part + d
    rows = pl.ds(pl.multiple_of(n * _NB, 8), _NB)

    @pl.when(i == 0)
    def _init():
        acc_ref[rows, :] = part

    @pl.when(i > 0)
    def _accum():
        acc_ref[rows, :] += part

    @pl.when((i == 6) & (n == pl.num_programs(1) - 1))
    def _epilogue():
        nrows = acc_ref.shape[0]
        nchunks = 5
        ch = nrows // nchunks
        inv_n = 1.0 / nrows

        def _bn_relu_to_h(scale, shift):
            for c in range(nchunks):
                r = pl.ds(c * ch, ch)
                h_ref[r, :] = jnp.maximum(
                    acc_ref[r, :] * scale + shift, 0.0
                ).astype(jnp.bfloat16)

        def _stats():
            s = None
            for c in range(nchunks):
                cs = jnp.sum(acc_ref[pl.ds(c * ch, ch), :], axis=0,
                             keepdims=True)
                s = cs if s is None else s + cs
            mean = s * inv_n
            v = None
            for c in range(nchunks):
                cv = jnp.sum(
                    (acc_ref[pl.ds(c * ch, ch), :] - mean) ** 2,
                    axis=0, keepdims=True)
                v = cv if v is None else v + cv
            return mean, v * inv_n

        mean1, var1 = _stats()
        scale1 = g1_ref[...] / jnp.sqrt(var1 + _EPS)
        _bn_relu_to_h(scale1, be1_ref[...] - mean1 * scale1)

        w2 = wdense_ref[...].astype(jnp.bfloat16)
        for c in range(2):
            r = pl.ds(c * (nrows // 2), nrows // 2)
            acc_ref[r, :] = jnp.dot(
                h_ref[r, :], w2, preferred_element_type=jnp.float32)

        mean2, var2 = _stats()
        scale2 = g2_ref[...] / jnp.sqrt(var2 + _EPS)
        _bn_relu_to_h(scale2, be2_ref[...] - mean2 * scale2)

        dims = (((1,), (1,)), ((), ()))
        wct = wct_ref[...].astype(jnp.bfloat16)
        wdt = wdt_ref[...].astype(jnp.bfloat16)
        bct = jnp.transpose(bc_ref[...])
        for c in range(2):
            r = pl.ds(c * (nrows // 2), nrows // 2)
            h2 = h_ref[r, :]
            logits_t = jax.lax.dot_general(
                wct, h2, dims, preferred_element_type=jnp.float32
            ) + bct
            logits_ref[:, r] = logits_t
            m = jnp.max(logits_t, axis=0, keepdims=True)
            e = jnp.exp(logits_t - m)
            probs_ref[:, r] = e / jnp.sum(e, axis=0, keepdims=True)
            deltas_ref[r, :] = jax.lax.dot_general(
                h2, wdt, dims, preferred_element_type=jnp.float32
            ) + bd_ref[...]


def kernel(pooled_rois, W1, b1, gamma1, beta1, W2, b2, gamma2, beta2, Wc, bc, Wd, bd):
    n = pooled_rois.shape[0]
    a_t = jnp.transpose(pooled_rois, (1, 2, 0, 3))
    wct = jnp.transpose(Wc)
    wdt = jnp.transpose(Wd)
    nc = Wc.shape[1]
    nd = Wd.shape[1]
    num_nb = n // _NB

    row = lambda v: v.reshape(1, -1)
    full = lambda arr: pl.BlockSpec(arr.shape, lambda i, j: (0,) * arr.ndim)

    logits, probs, deltas = pl.pallas_call(
        _bbox_head_kernel,
        grid=(7, num_nb),
        in_specs=[
            pl.BlockSpec((1, 7, 200, 256), lambda i, j: (i, 0, 5 * j, 0)),
            pl.BlockSpec((1, 7, 200, 256), lambda i, j: (i, 0, 5 * j + 1, 0)),
            pl.BlockSpec((1, 7, 200, 256), lambda i, j: (i, 0, 5 * j + 2, 0)),
            pl.BlockSpec((1, 7, 200, 256), lambda i, j: (i, 0, 5 * j + 3, 0)),
            pl.BlockSpec((1, 7, 200, 256), lambda i, j: (i, 0, 5 * j + 4, 0)),
            pl.BlockSpec((1, 7, 256, 1024), lambda i, j: (i, 0, 0, 0)),
            full(W2), full(wct), full(wdt),
            full(row(gamma1)), full(row(beta1)),
            full(row(gamma2)), full(row(beta2)),
            full(row(bc)), full(row(bd)),
        ],
        out_specs=[
            pl.BlockSpec((nc, n), lambda i, j: (0, 0)),
            pl.BlockSpec((nc, n), lambda i, j: (0, 0)),
            pl.BlockSpec((n, nd), lambda i, j: (0, 0)),
        ],
        out_shape=[
            jax.ShapeDtypeStruct((nc, n), jnp.float32),
            jax.ShapeDtypeStruct((nc, n), jnp.float32),
            jax.ShapeDtypeStruct((n, nd), jnp.float32),
        ],
        scratch_shapes=[
            pltpu.VMEM((n, 1024), jnp.float32),
            pltpu.VMEM((n, 1024), jnp.bfloat16),
        ],
        compiler_params=pltpu.CompilerParams(
            dimension_semantics=("arbitrary", "arbitrary"),
        ),
    )(
        a_t, a_t, a_t, a_t, a_t, W1, W2, wct, wdt,
        row(gamma1), row(beta1), row(gamma2), row(beta2),
        row(bc), row(bd),
    )
    return (
        jnp.transpose(logits),
        jnp.transpose(probs),
        deltas.reshape(n, _NUM_CLASSES, 4),
    )


# speedup vs baseline: 2.2246x; 1.0129x over previous
import jax
import jax.numpy as jnp
from jax.experimental import pallas as pl
from jax.experimental.pallas import tpu as pltpu

_NUM_CLASSES = 81
_EPS = 1e-3
_NB = 1000


def _bbox_head_kernel(
    a0_ref, a1_ref, a2_ref, a3_ref, a4_ref,
    w1_ref, wdense_ref, wct_ref, wdt_ref,
    g1_ref, be1_ref, g2_ref, be2_ref, bc_ref, bd_ref,
    logits_ref, probs_ref, deltas_ref,
    acc_ref, h_ref,
):
    i = pl.program_id(0)
    n = pl.program_id(1)
    a_refs = (a0_ref, a1_ref, a2_ref, a3_ref, a4_ref)
    part = None
    for jj in range(7):
        a = jnp.concatenate(
            [r[0, jj].astype(jnp.bfloat16) for r in a_refs], axis=0)
        w = w1_ref[0, jj].astype(jnp.bfloat16)
        d = jnp.dot(a, w, preferred_element_type=jnp.float32)
        part = d if part is None else part + d
    rows = pl.ds(pl.multiple_of(n * _NB, 8), _NB)

    @pl.when(i == 0)
    def _init():
        acc_ref[rows, :] = part

    @pl.when(i > 0)
    def _accum():
        acc_ref[rows, :] += part

    @pl.when((i == 6) & (n == pl.num_programs(1) - 1))
    def _epilogue():
        nrows = acc_ref.shape[0]
        nchunks = 5
        ch = nrows // nchunks
        inv_n = 1.0 / nrows

        def _bn_relu_to_h(scale, shift):
            for c in range(nchunks):
                r = pl.ds(c * ch, ch)
                h_ref[r, :] = jnp.maximum(
                    acc_ref[r, :] * scale + shift, 0.0
                ).astype(jnp.bfloat16)

        def _stats():
            s = sq = None
            for c in range(nchunks):
                blk = acc_ref[pl.ds(c * ch, ch), :]
                cs = jnp.sum(blk, axis=0, keepdims=True)
                cq = jnp.sum(blk * blk, axis=0, keepdims=True)
                s = cs if s is None else s + cs
                sq = cq if sq is None else sq + cq
            mean = s * inv_n
            return mean, sq * inv_n - mean * mean

        mean1, var1 = _stats()
        scale1 = g1_ref[...] / jnp.sqrt(var1 + _EPS)
        _bn_relu_to_h(scale1, be1_ref[...] - mean1 * scale1)

        w2 = wdense_ref[...].astype(jnp.bfloat16)
        for c in range(2):
            r = pl.ds(c * (nrows // 2), nrows // 2)
            acc_ref[r, :] = jnp.dot(
                h_ref[r, :], w2, preferred_element_type=jnp.float32)

        mean2, var2 = _stats()
        scale2 = g2_ref[...] / jnp.sqrt(var2 + _EPS)
        _bn_relu_to_h(scale2, be2_ref[...] - mean2 * scale2)

        dims = (((1,), (1,)), ((), ()))
        wct = wct_ref[...].astype(jnp.bfloat16)
        wdt = wdt_ref[...].astype(jnp.bfloat16)
        bct = jnp.transpose(bc_ref[...])
        for c in range(2):
            r = pl.ds(c * (nrows // 2), nrows // 2)
            h2 = h_ref[r, :]
            logits_t = jax.lax.dot_general(
                wct, h2, dims, preferred_element_type=jnp.float32
            ) + bct
            logits_ref[:, r] = logits_t
            m = jnp.max(logits_t, axis=0, keepdims=True)
            e = jnp.exp(logits_t - m)
            probs_ref[:, r] = e / jnp.sum(e, axis=0, keepdims=True)
            deltas_ref[r, :] = jax.lax.dot_general(
                h2, wdt, dims, preferred_element_type=jnp.float32
            ) + bd_ref[...]


def kernel(pooled_rois, W1, b1, gamma1, beta1, W2, b2, gamma2, beta2, Wc, bc, Wd, bd):
    n = pooled_rois.shape[0]
    a_t = jnp.transpose(pooled_rois, (1, 2, 0, 3))
    wct = jnp.transpose(Wc)
    wdt = jnp.transpose(Wd)
    nc = Wc.shape[1]
    nd = Wd.shape[1]
    num_nb = n // _NB

    row = lambda v: v.reshape(1, -1)
    full = lambda arr: pl.BlockSpec(arr.shape, lambda i, j: (0,) * arr.ndim)

    logits, probs, deltas = pl.pallas_call(
        _bbox_head_kernel,
        grid=(7, num_nb),
        in_specs=[
            pl.BlockSpec((1, 7, 200, 256), lambda i, j: (i, 0, 5 * j, 0)),
            pl.BlockSpec((1, 7, 200, 256), lambda i, j: (i, 0, 5 * j + 1, 0)),
            pl.BlockSpec((1, 7, 200, 256), lambda i, j: (i, 0, 5 * j + 2, 0)),
            pl.BlockSpec((1, 7, 200, 256), lambda i, j: (i, 0, 5 * j + 3, 0)),
            pl.BlockSpec((1, 7, 200, 256), lambda i, j: (i, 0, 5 * j + 4, 0)),
            pl.BlockSpec((1, 7, 256, 1024), lambda i, j: (i, 0, 0, 0)),
            full(W2), full(wct), full(wdt),
            full(row(gamma1)), full(row(beta1)),
            full(row(gamma2)), full(row(beta2)),
            full(row(bc)), full(row(bd)),
        ],
        out_specs=[
            pl.BlockSpec((nc, n), lambda i, j: (0, 0)),
            pl.BlockSpec((nc, n), lambda i, j: (0, 0)),
            pl.BlockSpec((n, nd), lambda i, j: (0, 0)),
        ],
        out_shape=[
            jax.ShapeDtypeStruct((nc, n), jnp.float32),
            jax.ShapeDtypeStruct((nc, n), jnp.float32),
            jax.ShapeDtypeStruct((n, nd), jnp.float32),
        ],
        scratch_shapes=[
            pltpu.VMEM((n, 1024), jnp.float32),
            pltpu.VMEM((n, 1024), jnp.bfloat16),
        ],
        compiler_params=pltpu.CompilerParams(
            dimension_semantics=("arbitrary", "arbitrary"),
        ),
    )(
        a_t, a_t, a_t, a_t, a_t, W1, W2, wct, wdt,
        row(gamma1), row(beta1), row(gamma2), row(beta2),
        row(bc), row(bd),
    )
    return (
        jnp.transpose(logits),
        jnp.transpose(probs),
        deltas.reshape(n, _NUM_CLASSES, 4),
    )


# speedup vs baseline: 2.2412x; 1.0075x over previous
import jax
import jax.numpy as jnp
from jax.experimental import pallas as pl
from jax.experimental.pallas import tpu as pltpu

_NUM_CLASSES = 81
_EPS = 1e-3
_NB = 1000


def _bbox_head_kernel(
    a0_ref, a1_ref, a2_ref, a3_ref, a4_ref,
    w1_ref, wdense_ref, wct_ref, wdt_ref,
    g1_ref, be1_ref, g2_ref, be2_ref, bc_ref, bd_ref,
    logits_ref, probs_ref, deltas_ref,
    acc_ref, h_ref,
):
    i = pl.program_id(0)
    n = pl.program_id(1)
    a_refs = (a0_ref, a1_ref, a2_ref, a3_ref, a4_ref)
    part = None
    for jj in range(7):
        a = jnp.concatenate(
            [r[0, jj].astype(jnp.bfloat16) for r in a_refs], axis=0)
        w = w1_ref[0, jj].astype(jnp.bfloat16)
        d = jnp.dot(a, w, preferred_element_type=jnp.float32)
        part = d if part is None else part + d
    rows = pl.ds(pl.multiple_of(n * _NB, 8), _NB)

    @pl.when(i == 0)
    def _init():
        acc_ref[rows, :] = part

    @pl.when(i > 0)
    def _accum():
        acc_ref[rows, :] += part

    @pl.when((i == 6) & (n == pl.num_programs(1) - 1))
    def _epilogue():
        nrows = acc_ref.shape[0]
        nchunks = 5
        ch = nrows // nchunks
        inv_n = 1.0 / nrows

        def _bn_relu_to_h(scale, shift):
            for c in range(nchunks):
                r = pl.ds(c * ch, ch)
                h_ref[r, :] = jnp.maximum(
                    acc_ref[r, :] * scale + shift, 0.0
                ).astype(jnp.bfloat16)

        def _stats():
            s = sq = None
            for c in range(nchunks):
                blk = acc_ref[pl.ds(c * ch, ch), :]
                cs = jnp.sum(blk, axis=0, keepdims=True)
                cq = jnp.sum(blk * blk, axis=0, keepdims=True)
                s = cs if s is None else s + cs
                sq = cq if sq is None else sq + cq
            mean = s * inv_n
            return mean, sq * inv_n - mean * mean

        mean1, var1 = _stats()
        scale1 = g1_ref[...] / jnp.sqrt(var1 + _EPS)
        _bn_relu_to_h(scale1, be1_ref[...] - mean1 * scale1)

        w2 = wdense_ref[...].astype(jnp.bfloat16)
        for c in range(2):
            r = pl.ds(c * (nrows // 2), nrows // 2)
            acc_ref[r, :] = jnp.dot(
                h_ref[r, :], w2, preferred_element_type=jnp.float32)

        mean2, var2 = _stats()
        scale2 = g2_ref[...] / jnp.sqrt(var2 + _EPS)
        _bn_relu_to_h(scale2, be2_ref[...] - mean2 * scale2)

        dims = (((1,), (1,)), ((), ()))
        wct = wct_ref[...].astype(jnp.bfloat16)
        wdt = wdt_ref[...].astype(jnp.bfloat16)
        bct = jnp.transpose(bc_ref[...])
        bdt = jnp.transpose(bd_ref[...])
        for c in range(2):
            r = pl.ds(c * (nrows // 2), nrows // 2)
            h2 = h_ref[r, :]
            logits_t = jax.lax.dot_general(
                wct, h2, dims, preferred_element_type=jnp.float32
            ) + bct
            logits_ref[:, r] = logits_t
            m = jnp.max(logits_t, axis=0, keepdims=True)
            e = jnp.exp(logits_t - m)
            probs_ref[:, r] = e / jnp.sum(e, axis=0, keepdims=True)
            deltas_ref[:, r] = jax.lax.dot_general(
                wdt, h2, dims, preferred_element_type=jnp.float32
            ) + bdt


def kernel(pooled_rois, W1, b1, gamma1, beta1, W2, b2, gamma2, beta2, Wc, bc, Wd, bd):
    n = pooled_rois.shape[0]
    a_t = jnp.transpose(pooled_rois, (1, 2, 0, 3))
    wct = jnp.transpose(Wc)
    wdt = jnp.transpose(Wd)
    nc = Wc.shape[1]
    nd = Wd.shape[1]
    num_nb = n // _NB

    row = lambda v: v.reshape(1, -1)
    full = lambda arr: pl.BlockSpec(arr.shape, lambda i, j: (0,) * arr.ndim)

    logits, probs, deltas = pl.pallas_call(
        _bbox_head_kernel,
        grid=(7, num_nb),
        in_specs=[
            pl.BlockSpec((1, 7, 200, 256), lambda i, j: (i, 0, 5 * j, 0)),
            pl.BlockSpec((1, 7, 200, 256), lambda i, j: (i, 0, 5 * j + 1, 0)),
            pl.BlockSpec((1, 7, 200, 256), lambda i, j: (i, 0, 5 * j + 2, 0)),
            pl.BlockSpec((1, 7, 200, 256), lambda i, j: (i, 0, 5 * j + 3, 0)),
            pl.BlockSpec((1, 7, 200, 256), lambda i, j: (i, 0, 5 * j + 4, 0)),
            pl.BlockSpec((1, 7, 256, 1024), lambda i, j: (i, 0, 0, 0)),
            full(W2), full(wct), full(wdt),
            full(row(gamma1)), full(row(beta1)),
            full(row(gamma2)), full(row(beta2)),
            full(row(bc)), full(row(bd)),
        ],
        out_specs=[
            pl.BlockSpec((nc, n), lambda i, j: (0, 0)),
            pl.BlockSpec((nc, n), lambda i, j: (0, 0)),
            pl.BlockSpec((nd, n), lambda i, j: (0, 0)),
        ],
        out_shape=[
            jax.ShapeDtypeStruct((nc, n), jnp.float32),
            jax.ShapeDtypeStruct((nc, n), jnp.float32),
            jax.ShapeDtypeStruct((nd, n), jnp.float32),
        ],
        scratch_shapes=[
            pltpu.VMEM((n, 1024), jnp.float32),
            pltpu.VMEM((n, 1024), jnp.bfloat16),
        ],
        compiler_params=pltpu.CompilerParams(
            dimension_semantics=("arbitrary", "arbitrary"),
        ),
    )(
        a_t, a_t, a_t, a_t, a_t, W1, W2, wct, wdt,
        row(gamma1), row(beta1), row(gamma2), row(beta2),
        row(bc), row(bd),
    )
    return (
        jnp.transpose(logits),
        jnp.transpose(probs),
        jnp.transpose(deltas).reshape(n, _NUM_CLASSES, 4),
    )
